# Initial kernel scaffold; baseline (speedup 1.0000x reference)
import jax
import jax.numpy as jnp
from jax.experimental import pallas as pl


def kernel(feats, edge_index, W0, b0, W1, b1, W2, b2, W3, b3, lin1_W, lin1_b, lin2_W, lin2_b):
    raise NotImplementedError("write your pallas kernel here")


# speedup vs baseline: 8.4969x; 8.4969x over previous
import jax
import jax.numpy as jnp
from jax import lax
from jax.experimental import pallas as pl
from jax.experimental.pallas import tpu as pltpu
from jax.experimental.pallas import tpu_sc as plsc

N = 100000
E = 1600000
B = 4096
ERP = 12544
GRP = 8
NGS = 98
NP = 100096
NSUB = NP // 16
ZCH = 368
PK = NP * 32 // 128
BF = PK // 8

_mesh = plsc.VectorSubcoreMesh(core_axis_name="c", subcore_axis_name="s")


def _deg_body(ei, deg_out, dacc, zbuf, onesb, idxb):
    c = lax.axis_index("c")
    s = lax.axis_index("s")
    z16 = jnp.zeros((16,), jnp.float32)
    o16 = jnp.ones((16,), jnp.float32)

    def fill_z(i, _):
        zbuf[pl.ds(i * 16, 16)] = z16
        return 0
    lax.fori_loop(0, NSUB // 16, fill_z, 0)

    def fill_o(i, _):
        onesb[pl.ds(i * 16, 16)] = o16
        return 0
    lax.fori_loop(0, 8, fill_o, 0)

    pltpu.sync_copy(zbuf, dacc.at[pl.ds(s * NSUB, NSUB)])
    plsc.subcore_barrier()

    base = c * ERP + s * NGS * GRP

    def grp(g, _):
        pltpu.sync_copy(ei.at[pl.ds(base + g * GRP, GRP)], idxb)
        for j in range(GRP):
            pltpu.sync_copy(onesb, dacc.at[idxb.at[j]], add=True)
        return 0
    lax.fori_loop(0, NGS, grp, 0)

    plsc.subcore_barrier()
    pltpu.sync_copy(dacc.at[pl.ds(s * NSUB, NSUB)],
                    deg_out.at[pl.ds(c * NP + s * NSUB, NSUB)])


_deg_call = pl.kernel(
    _deg_body,
    out_type=jax.ShapeDtypeStruct((2 * NP,), jnp.float32),
    mesh=_mesh,
    compiler_params=pltpu.CompilerParams(use_tc_tiling_on_sc=False),
    scratch_types=[
        pltpu.VMEM_SHARED((NP,), jnp.float32),
        pltpu.VMEM((NSUB,), jnp.float32),
        pltpu.VMEM((128,), jnp.float32),
        pltpu.VMEM((GRP, 128), jnp.int32),
    ],
)


def _msg_body(table, ei, agg_out, acc, zbuf, srcb, dstb, adjb, rows, sem):
    c = lax.axis_index("c")
    s = lax.axis_index("s")
    z16 = jnp.zeros((16,), jnp.float32)

    def fill_z(i, _):
        zbuf[i, :] = z16
        return 0
    lax.fori_loop(0, ZCH, fill_z, 0)

    def zero_acc(k, _):
        pltpu.sync_copy(zbuf, acc.at[pl.ds(s * NSUB + k * ZCH, ZCH)])
        return 0
    lax.fori_loop(0, NSUB // ZCH, zero_acc, 0)
    plsc.subcore_barrier()

    base = s * NGS * GRP

    def grp(g, _):
        r0 = base + g * GRP
        pltpu.sync_copy(ei.at[pl.ds(r0, GRP)], srcb)
        pltpu.sync_copy(ei.at[pl.ds(ERP + r0, GRP)], dstb)

        def adj(k, _):
            sl = pl.ds(k * 16, 16)
            for j in range(GRP):
                adjb[j, sl] = srcb[j, sl] * 2 + c
            return 0
        lax.fori_loop(0, 8, adj, 0)

        cps = [pltpu.async_copy(table.at[adjb.at[j]], rows.at[j], sem)
               for j in range(GRP)]
        for cp in cps:
            cp.wait()
        for j in range(GRP):
            pltpu.sync_copy(rows.at[j], acc.at[dstb.at[j]], add=True)
        return 0
    lax.fori_loop(0, NGS, grp, 0)

    plsc.subcore_barrier()
    pltpu.sync_copy(acc.at[pl.ds(s * NSUB, NSUB)],
                    agg_out.at[pl.ds(c * NP + s * NSUB, NSUB)])


_msg_call = pl.kernel(
    _msg_body,
    out_type=jax.ShapeDtypeStruct((2 * NP, 16), jnp.float32),
    mesh=_mesh,
    compiler_params=pltpu.CompilerParams(use_tc_tiling_on_sc=False),
    scratch_types=[
        pltpu.VMEM_SHARED((NP, 16), jnp.float32),
        pltpu.VMEM((ZCH, 16), jnp.float32),
        pltpu.VMEM((GRP, 128), jnp.int32),
        pltpu.VMEM((GRP, 128), jnp.int32),
        pltpu.VMEM((GRP, 128), jnp.int32),
        pltpu.VMEM((GRP, 128, 16), jnp.float32),
        pltpu.SemaphoreType.DMA,
    ],
)


def _prep_body(od_ref, id_ref, ns_ref, nd_ref):
    ns_ref[...] = lax.rsqrt(jnp.maximum(od_ref[...], 1.0))
    nd_ref[...] = lax.rsqrt(jnp.maximum(id_ref[...], 1.0))


_prep_call = pl.pallas_call(
    _prep_body,
    out_shape=[
        jax.ShapeDtypeStruct((NP // 128, 128), jnp.float32),
        jax.ShapeDtypeStruct((NP // 128, 128), jnp.float32),
    ],
)


def _scale_body(f_ref, ns_ref, t_ref):
    t_ref[...] = f_ref[...] * ns_ref[...]


_scale_call = pl.pallas_call(
    _scale_body,
    grid=(4,),
    in_specs=[
        pl.BlockSpec((PK // 4, 128), lambda i: (i, 0)),
        pl.BlockSpec((PK // 4, 128), lambda i: (i, 0)),
    ],
    out_specs=pl.BlockSpec((PK // 4, 128), lambda i: (i, 0)),
    out_shape=jax.ShapeDtypeStruct((PK, 128), jnp.float32),
)


def _dense_body(a_ref, w_ref, b_ref, nd_ref, ns_ref, x_ref, t_ref):
    h = jnp.dot(a_ref[...], w_ref[...], preferred_element_type=jnp.float32, precision=lax.Precision.HIGHEST)
    x = jnp.tanh(h * nd_ref[...] + b_ref[...])
    x_ref[...] = x
    t_ref[...] = x * ns_ref[...]


_dense_call = pl.pallas_call(
    _dense_body,
    grid=(8,),
    in_specs=[
        pl.BlockSpec((BF, 128), lambda i: (i, 0)),
        pl.BlockSpec((128, 128), lambda i: (0, 0)),
        pl.BlockSpec((1, 128), lambda i: (0, 0)),
        pl.BlockSpec((BF, 128), lambda i: (i, 0)),
        pl.BlockSpec((BF, 128), lambda i: (i, 0)),
    ],
    out_specs=[
        pl.BlockSpec((BF, 128), lambda i: (i, 0)),
        pl.BlockSpec((BF, 128), lambda i: (i, 0)),
    ],
    out_shape=[
        jax.ShapeDtypeStruct((PK, 128), jnp.float32),
        jax.ShapeDtypeStruct((PK, 128), jnp.float32),
    ],
)


def _mlp_body(x0, x1, x2, x3, w1, b1, w2, b2, o_ref):
    acc = jnp.zeros((B, 32), jnp.float32) + b1[...]
    w1v = w1[...]
    for l, x in enumerate((x0, x1, x2, x3)):
        xv = x[...]
        u = xv[:B]
        v = xv[B:]
        acc = acc + jnp.dot(u, w1v[l * 32:(l + 1) * 32],
                            preferred_element_type=jnp.float32, precision=lax.Precision.HIGHEST)
        acc = acc + jnp.dot(v, w1v[128 + l * 32:128 + (l + 1) * 32],
                            preferred_element_type=jnp.float32, precision=lax.Precision.HIGHEST)
    h = jnp.maximum(acc, 0.0)
    o_ref[...] = jnp.dot(h, w2[...], preferred_element_type=jnp.float32, precision=lax.Precision.HIGHEST) + b2[...]


_mlp_call = pl.pallas_call(
    _mlp_body,
    out_shape=jax.ShapeDtypeStruct((B, 1), jnp.float32),
)


def _blockdiag4(W):
    Z = jnp.zeros((128, 128), jnp.float32)
    for k in range(4):
        Z = Z.at[k * 32:(k + 1) * 32, k * 32:(k + 1) * 32].set(W)
    return Z


def kernel(feats, edge_index, W0, b0, W1, b1, W2, b2, W3, b3,
           lin1_W, lin1_b, lin2_W, lin2_b):
    ei_p = jnp.pad(edge_index.reshape(2, E // 128, 128),
                   ((0, 0), (0, ERP - E // 128), (0, 0)),
                   constant_values=N).reshape(2 * ERP, 128)

    deg = _deg_call(ei_p)
    ns2, nd2 = _prep_call(deg[:NP].reshape(NP // 128, 128),
                          deg[NP:].reshape(NP // 128, 128))
    ns_rep = jnp.broadcast_to(ns2.reshape(NP, 1), (NP, 32)).reshape(PK, 128)
    nd_rep = jnp.broadcast_to(nd2.reshape(NP, 1), (NP, 32)).reshape(PK, 128)

    fpad = jnp.pad(feats, ((0, NP - N), (0, 28))).reshape(PK, 128)
    table = _scale_call(fpad, ns_rep)

    W0p = jnp.pad(W0, ((0, 28), (0, 0)))
    heads = []
    for Wl, bl in ((W0p, b0), (W1, b1), (W2, b2), (W3, b3)):
        agg = _msg_call(table.reshape(2 * NP, 16), ei_p)
        apk = agg.reshape(2, NP, 16).transpose(1, 0, 2).reshape(PK, 128)
        x, table = _dense_call(apk, _blockdiag4(Wl),
                               jnp.tile(bl, 4).reshape(1, 128),
                               nd_rep, ns_rep)
        heads.append(x[:2 * B * 32 // 128].reshape(2 * B, 32))

    out = _mlp_call(heads[0], heads[1], heads[2], heads[3],
                    lin1_W, lin1_b.reshape(1, 32),
                    lin2_W, lin2_b.reshape(1, 1))
    return out.reshape(B)
